# baseline (speedup 1.0000x reference)
import functools

import jax
import jax.numpy as jnp
from jax import lax
from jax.experimental import pallas as pl
from jax.experimental.pallas import tpu as pltpu
from jax.experimental.pallas import tpu_sc as plsc

_VOCAB = 1000000
_EMBED = 64
_HIDDEN = 128
_OUT = 50
_SEQ = 200
_BATCH = 4096

_NC = 2
_NS = 16
_L = 16
_NW = _NC * _NS
_BPW = _BATCH // _NW
_HALF = _SEQ // 2
_UN = 8
_NOUT = _BPW // _UN


def _sc_pooled_sums(xt, emb):
    mesh = plsc.VectorSubcoreMesh(
        core_axis_name="c", subcore_axis_name="s",
        num_cores=_NC, num_subcores=_NS)

    @functools.partial(
        pl.kernel,
        out_type=jax.ShapeDtypeStruct((_BATCH, _EMBED), jnp.float32),
        mesh=mesh,
        scratch_types=[
            pltpu.VMEM((2 * _BPW, _HALF), jnp.int32),
            pltpu.VMEM((2, 2, _HALF, _EMBED), jnp.float32),
            pltpu.VMEM((_BPW, _EMBED), jnp.float32),
            pltpu.SemaphoreType.DMA,
            pltpu.SemaphoreType.DMA,
        ],
        compiler_params=pltpu.CompilerParams(use_tc_tiling_on_sc=False),
    )
    def body(xt_hbm, emb_hbm, out_hbm, idx_v, gbuf, acc_v, sem0, sem1):
        wid = lax.axis_index("s") * _NC + lax.axis_index("c")
        base = wid * _BPW
        pltpu.sync_copy(xt_hbm.at[pl.ds(2 * base, 2 * _BPW)], idx_v)
        sems = (sem0, sem1)

        def fire(b, slot):
            pltpu.async_copy(emb_hbm.at[idx_v.at[2 * b]], gbuf.at[slot, 0], sems[slot])
            pltpu.async_copy(emb_hbm.at[idx_v.at[2 * b + 1]], gbuf.at[slot, 1], sems[slot])

        def drain(slot):
            pltpu.make_async_copy(emb_hbm.at[idx_v.at[0]], gbuf.at[slot, 0], sems[slot]).wait()
            pltpu.make_async_copy(emb_hbm.at[idx_v.at[1]], gbuf.at[slot, 1], sems[slot]).wait()

        fire(0, 0)

        def outer(o, carry):
            for j in range(_UN):
                b = o * _UN + j
                slot = j % 2

                @pl.when(b + 1 < _BPW)
                def _():
                    fire(b + 1, (j + 1) % 2)

                drain(slot)
                b0 = gbuf.at[slot, 0]
                b1 = gbuf.at[slot, 1]

                def accum(r, vs, b0=b0, b1=b1):
                    out = []
                    for k in range(_EMBED // _L):
                        v = vs[k] + b0[r, pl.ds(k * _L, _L)]
                        out.append(v + b1[r, pl.ds(k * _L, _L)])
                    return tuple(out)

                zero = jnp.zeros((_L,), jnp.float32)
                acc = lax.fori_loop(0, _HALF, accum, (zero,) * (_EMBED // _L))
                for k in range(_EMBED // _L):
                    acc_v[b, pl.ds(k * _L, _L)] = acc[k]
            return carry

        lax.fori_loop(0, _NOUT, outer, jnp.int32(0))
        pltpu.sync_copy(acc_v, out_hbm.at[pl.ds(base, _BPW)])

    return body(xt, emb)


def _tc_mlp(pooled_sums, W1, b1, W2, b2):
    def body(p_ref, w1_ref, b1_ref, w2_ref, b2_ref, o_ref):
        p = p_ref[...] * (1.0 / _SEQ)
        h = jnp.dot(p, w1_ref[...], preferred_element_type=jnp.float32) + b1_ref[...]
        z = jnp.dot(h, w2_ref[...], preferred_element_type=jnp.float32) + b2_ref[...]
        z = z - jnp.max(z, axis=-1, keepdims=True)
        e = jnp.exp(z)
        o_ref[...] = e / jnp.sum(e, axis=-1, keepdims=True)

    return pl.pallas_call(
        body,
        out_shape=jax.ShapeDtypeStruct((_BATCH, _OUT), jnp.float32),
    )(pooled_sums, W1, b1.reshape(1, _HIDDEN), W2, b2.reshape(1, _OUT))


def kernel(x, emb, W1, b1, W2, b2):
    xt = jnp.transpose(x).reshape(2 * _BATCH, _HALF)
    pooled_sums = _sc_pooled_sums(xt, emb)
    return _tc_mlp(pooled_sums, W1, b1, W2, b2)
